# baseline (speedup 1.0000x reference)
import functools

import jax
import jax.numpy as jnp
from jax import lax
from jax.experimental import pallas as pl
from jax.experimental.pallas import tpu as pltpu
from jax.experimental.pallas import tpu_sc as plsc

N = 50000
NPAD = 50048
E = 800000
NW = 32
EPW = E // NW
CH = 125
NCH = EPW // CH
RPT = NPAD // 16
RCH = 136
NRCH = RPT // RCH


NBUF = 8
LOOK = 4


def _seg_sum_body(table_hbm, src_hbm, dst_hbm, zero_hbm, out_hbm,
                  src_v, dst_v, bufs, acc, gsems, ssems, zsem, W):
    c = lax.axis_index("c")
    s = lax.axis_index("s")
    w = c * 16 + s

    pltpu.async_copy(zero_hbm, acc.at[pl.ds(s * RPT, RPT)], zsem)
    pltpu.sync_copy(src_hbm.at[w], src_v)
    pltpu.sync_copy(dst_hbm.at[w], dst_v)
    pltpu.make_async_copy(zero_hbm, acc.at[pl.ds(s * RPT, RPT)], zsem).wait()
    plsc.subcore_barrier()

    def _g_start(j, b):
        pltpu.async_copy(table_hbm.at[src_v.at[j]], bufs[b], gsems[b])

    def _g_wait(b):
        pltpu.make_async_copy(table_hbm.at[src_v.at[0]], bufs[b],
                              gsems[b]).wait()

    def _s_start(j, b):
        pltpu.async_copy(bufs[b], acc.at[dst_v.at[j]], ssems[b], add=True)

    def _s_wait(b):
        pltpu.make_async_copy(bufs[b], acc.at[dst_v.at[0]], ssems[b]).wait()

    for j in range(LOOK):
        _g_start(j, j)

    def _body(i, _):
        for off in range(NBUF):
            j = NBUF * i + off
            b = off
            bn = (off + LOOK) % NBUF
            _g_wait(b)
            _s_start(j, b)

            @pl.when(j + LOOK < NCH)
            def _():
                @pl.when(j >= LOOK)
                def _():
                    _s_wait(bn)
                _g_start(j + LOOK, bn)
        return 0
    lax.fori_loop(0, NCH // NBUF, _body, 0)
    for b in range(NBUF):
        _s_wait(b)
    plsc.subcore_barrier()

    r = s * RPT
    pltpu.async_copy(acc.at[pl.ds(r, RPT)], out_hbm.at[c, pl.ds(r, RPT)],
                     zsem)
    pltpu.make_async_copy(acc.at[pl.ds(r, RPT)],
                          out_hbm.at[c, pl.ds(r, RPT)], zsem).wait()


def _make_seg_sum(W):
    mesh = plsc.VectorSubcoreMesh(core_axis_name="c", subcore_axis_name="s")

    @functools.partial(
        pl.kernel,
        out_type=jax.ShapeDtypeStruct((2, NPAD, W), jnp.float32),
        mesh=mesh,
        compiler_params=pltpu.CompilerParams(use_tc_tiling_on_sc=False),
        scratch_types=[
            pltpu.VMEM((NCH, CH), jnp.int32),
            pltpu.VMEM((NCH, CH), jnp.int32),
        ] + [pltpu.VMEM((CH, W), jnp.float32) for _ in range(NBUF)]
        + [pltpu.VMEM_SHARED((NPAD, W), jnp.float32)]
        + [pltpu.SemaphoreType.DMA for _ in range(2 * NBUF + 1)],
    )
    def seg_sum(table_hbm, src_hbm, dst_hbm, zero_hbm, out_hbm,
                src_v, dst_v, *rest):
        bufs = rest[:NBUF]
        acc = rest[NBUF]
        gsems = rest[NBUF + 1:2 * NBUF + 1]
        ssems = rest[2 * NBUF + 1:3 * NBUF + 1]
        zsem = rest[3 * NBUF + 1]
        _seg_sum_body(table_hbm, src_hbm, dst_hbm, zero_hbm, out_hbm,
                      src_v, dst_v, bufs, acc, gsems, ssems, zsem, W)

    return seg_sum


_seg_sum_16 = _make_seg_sum(16)

_B = 2944
_BP = _B // 8
_NP8 = NPAD // 8


def _stage_mid_body(pa_ref, pb_ref, xa_ref, xb_ref, scnt_ref, sbc_ref,
                    wla_ref, wlb_ref, wra_ref, wrb_ref, b768_ref,
                    apk_ref, bpk_ref, c128_ref, y1_ref, zrc_ref, rcp_ref):
    f32 = jnp.float32
    agg_a = pa_ref[0] + pa_ref[1]
    agg_b = pb_ref[0] + pb_ref[1]
    cnt = jnp.dot(agg_b, scnt_ref[...], preferred_element_type=f32)
    rcp = 1.0 / jnp.maximum(cnt, 1.0)
    rcpf = jnp.dot(rcp, sbc_ref[...], preferred_element_type=f32)
    h = (jnp.dot(agg_a * rcpf, wla_ref[...], preferred_element_type=f32)
         + jnp.dot(agg_b * rcpf, wlb_ref[...], preferred_element_type=f32)
         + jnp.dot(xa_ref[...], wra_ref[...], preferred_element_type=f32)
         + jnp.dot(xb_ref[...], wrb_ref[...], preferred_element_type=f32)
         + b768_ref[...])
    h = jnp.maximum(h, 0.0)
    y1_ref[...] = jnp.dot(h, apk_ref[...], preferred_element_type=f32)
    zrc_ref[...] = jnp.dot(h, bpk_ref[...],
                           preferred_element_type=f32) + c128_ref[...]
    rcp_ref[...] = rcpf


def _stage_mid(parts_a, parts_b, xap, xbp, scnt, sbc, wla, wlb, wra, wrb,
               b768, apk, bpk, c128):
    blk = pl.BlockSpec((_BP, 128), lambda i: (i, 0))
    full = lambda r, c: pl.BlockSpec((r, c), lambda i: (0, 0))
    return pl.pallas_call(
        _stage_mid_body,
        grid=(NPAD // _B,),
        in_specs=[
            pl.BlockSpec((2, _BP, 128), lambda i: (0, i, 0)),
            pl.BlockSpec((2, _BP, 128), lambda i: (0, i, 0)),
            blk, blk,
            full(128, 8), full(8, 128),
            full(128, 768), full(128, 768), full(128, 768), full(128, 768),
            full(1, 768), full(768, 128), full(768, 128), full(1, 128),
        ],
        out_specs=[blk, blk, blk],
        out_shape=[
            jax.ShapeDtypeStruct((_NP8, 128), jnp.float32),
            jax.ShapeDtypeStruct((_NP8, 128), jnp.float32),
            jax.ShapeDtypeStruct((_NP8, 128), jnp.float32),
        ],
    )(parts_a, parts_b, xap, xbp, scnt, sbc, wla, wlb, wra, wrb,
      b768, apk, bpk, c128)


def _stage_out_body(parts_ref, zrc_ref, rcp_ref, f2_ref, b192_ref, out_ref):
    agg = parts_ref[0] + parts_ref[1]
    z = jnp.maximum(agg * rcp_ref[...] + zrc_ref[...], 0.0)
    out_ref[...] = jnp.dot(z, f2_ref[...],
                           preferred_element_type=jnp.float32) + b192_ref[...]


def _stage_out(parts2, zrc, rcp, f2p, b192):
    blk = pl.BlockSpec((_BP, 128), lambda i: (i, 0))
    return pl.pallas_call(
        _stage_out_body,
        grid=(NPAD // _B,),
        in_specs=[
            pl.BlockSpec((2, _BP, 128), lambda i: (0, i, 0)),
            blk, blk,
            pl.BlockSpec((128, 192), lambda i: (0, 0)),
            pl.BlockSpec((1, 192), lambda i: (0, 0)),
        ],
        out_specs=pl.BlockSpec((_BP, 192), lambda i: (i, 0)),
        out_shape=jax.ShapeDtypeStruct((_NP8, 192), jnp.float32),
    )(parts2, zrc, rcp, f2p, b192)


def kernel(x, edge_index, Wl1, bl1, Wr1, Wl2, bl2, Wr2,
           fc1_W, fc1_b, fc2_W, fc2_b):
    f32 = jnp.float32
    x24 = x.reshape(N, 24)
    padp = ((0, _NP8 - N // 8), (0, 0))
    xap = jnp.pad(x24[:, :16].reshape(N // 8, 128), padp)
    xbp = jnp.pad(jnp.concatenate(
        [x24[:, 16:], jnp.ones((N, 1), f32), jnp.zeros((N, 7), f32)],
        axis=1).reshape(N // 8, 128), padp)
    xp_a = xap.reshape(NPAD, 16)
    xp_b = xbp.reshape(NPAD, 16)
    src3 = edge_index[0].reshape(NW, NCH, CH)
    dst3 = edge_index[1].reshape(NW, NCH, CH)

    eye8 = jnp.eye(8, dtype=f32)
    eye12 = jnp.eye(12, dtype=f32)
    wl = jnp.einsum('st,gf->sgtf', eye12, Wl1).reshape(24, 96)
    wr = jnp.einsum('st,gf->sgtf', eye12, Wr1).reshape(24, 96)
    pad16 = lambda m: jnp.pad(m, ((0, 8), (0, 0)))
    wla = jnp.kron(eye8, wl[:16])
    wlb = jnp.kron(eye8, pad16(wl[16:]))
    wra = jnp.kron(eye8, wr[:16])
    wrb = jnp.kron(eye8, pad16(wr[16:]))
    b768 = jnp.tile(jnp.tile(bl1, 12), 8).reshape(1, 768)
    w1r = fc1_W.reshape(2, 12, 8, 8)
    a_f = jnp.einsum('gf,isfo->sgio', Wl2, w1r).reshape(96, 16)
    b_f = jnp.einsum('gf,isfo->sgio', Wr2, w1r).reshape(96, 16)
    apk = jnp.kron(eye8, a_f)
    bpk = jnp.kron(eye8, b_f)
    c16 = (jnp.einsum('f,isfo->io', bl2, w1r) + fc1_b).reshape(16)
    c128 = jnp.tile(c16, 8).reshape(1, 128)
    f2 = jnp.zeros((16, 24), f32)
    f2 = f2.at[:8, :12].set(fc2_W[0]).at[8:, 12:].set(fc2_W[1])
    f2p = jnp.kron(eye8, f2)
    b192 = jnp.tile(jnp.concatenate([fc2_b[0], fc2_b[1]]), 8).reshape(1, 192)
    scnt = jnp.zeros((128, 8), f32).at[jnp.arange(8) * 16 + 8,
                                       jnp.arange(8)].set(1.0)
    sbc = jnp.kron(eye8, jnp.ones((1, 16), f32))

    zr = jnp.zeros((RPT, 16), f32)
    parts1a = _seg_sum_16(xp_a, src3, dst3, zr)
    parts1b = _seg_sum_16(xp_b, src3, dst3, zr)
    y1p, zrc, rcp = _stage_mid(parts1a.reshape(2, _NP8, 128),
                               parts1b.reshape(2, _NP8, 128),
                               xap, xbp, scnt, sbc, wla, wlb, wra, wrb,
                               b768, apk, bpk, c128)
    y1 = y1p.reshape(NPAD, 16)
    parts2 = _seg_sum_16(y1, src3, dst3, zr)
    op = _stage_out(parts2.reshape(2, _NP8, 128), zrc, rcp, f2p, b192)
    o24 = op.reshape(NPAD, 24)[:N]
    return o24.reshape(N, 2, 12).transpose(1, 0, 2)


# speedup vs baseline: 1.1948x; 1.1948x over previous
import functools

import jax
import jax.numpy as jnp
from jax import lax
from jax.experimental import pallas as pl
from jax.experimental.pallas import tpu as pltpu
from jax.experimental.pallas import tpu_sc as plsc

N = 50000
NPAD = 50048
E = 800000
NW = 32
EPW = E // NW
CH = 125
NCH = EPW // CH
RPT = NPAD // 16
RCH = 136
NRCH = RPT // RCH


NBUF = 8
LOOK = 4


def _seg_sum_body(table_hbm, src_hbm, dst_hbm, zero_hbm, out_hbm,
                  src_v, dst_v, bufs, acc, gsems, ssems, zsem, W):
    c = lax.axis_index("c")
    s = lax.axis_index("s")
    w = c * 16 + s

    pltpu.async_copy(zero_hbm, acc.at[pl.ds(s * RPT, RPT)], zsem)
    pltpu.sync_copy(src_hbm.at[w], src_v)
    pltpu.sync_copy(dst_hbm.at[w], dst_v)
    pltpu.make_async_copy(zero_hbm, acc.at[pl.ds(s * RPT, RPT)], zsem).wait()
    plsc.subcore_barrier()

    def _g_start(j, b):
        pltpu.async_copy(table_hbm.at[src_v.at[j]], bufs[b], gsems[b])

    def _g_wait(b):
        pltpu.make_async_copy(table_hbm.at[src_v.at[0]], bufs[b],
                              gsems[b]).wait()

    def _s_start(j, b):
        pltpu.async_copy(bufs[b], acc.at[dst_v.at[j]], ssems[b], add=True)

    def _s_wait(b):
        pltpu.make_async_copy(bufs[b], acc.at[dst_v.at[0]], ssems[b]).wait()

    for j in range(LOOK):
        _g_start(j, j)

    def _body(i, _):
        for off in range(NBUF):
            j = NBUF * i + off
            b = off
            bn = (off + LOOK) % NBUF
            _g_wait(b)
            _s_start(j, b)

            @pl.when(j + LOOK < NCH)
            def _():
                @pl.when(j >= LOOK)
                def _():
                    _s_wait(bn)
                _g_start(j + LOOK, bn)
        return 0
    lax.fori_loop(0, NCH // NBUF, _body, 0)
    for b in range(NBUF):
        _s_wait(b)
    plsc.subcore_barrier()

    r = s * RPT
    pltpu.async_copy(acc.at[pl.ds(r, RPT)], out_hbm.at[c, pl.ds(r, RPT)],
                     zsem)
    pltpu.make_async_copy(acc.at[pl.ds(r, RPT)],
                          out_hbm.at[c, pl.ds(r, RPT)], zsem).wait()


def _make_seg_sum(W):
    mesh = plsc.VectorSubcoreMesh(core_axis_name="c", subcore_axis_name="s")

    @functools.partial(
        pl.kernel,
        out_type=jax.ShapeDtypeStruct((2, NPAD, W), jnp.float32),
        mesh=mesh,
        compiler_params=pltpu.CompilerParams(use_tc_tiling_on_sc=False),
        scratch_types=[
            pltpu.VMEM((NCH, CH), jnp.int32),
            pltpu.VMEM((NCH, CH), jnp.int32),
        ] + [pltpu.VMEM((CH, W), jnp.float32) for _ in range(NBUF)]
        + [pltpu.VMEM_SHARED((NPAD, W), jnp.float32)]
        + [pltpu.SemaphoreType.DMA for _ in range(2 * NBUF + 1)],
    )
    def seg_sum(table_hbm, src_hbm, dst_hbm, zero_hbm, out_hbm,
                src_v, dst_v, *rest):
        bufs = rest[:NBUF]
        acc = rest[NBUF]
        gsems = rest[NBUF + 1:2 * NBUF + 1]
        ssems = rest[2 * NBUF + 1:3 * NBUF + 1]
        zsem = rest[3 * NBUF + 1]
        _seg_sum_body(table_hbm, src_hbm, dst_hbm, zero_hbm, out_hbm,
                      src_v, dst_v, bufs, acc, gsems, ssems, zsem, W)

    return seg_sum


_seg_sum_16 = _make_seg_sum(16)

_B = 2944
_BP = _B // 8
_NP8 = NPAD // 8


def _stage_mid_body(pa_ref, pb_ref, xa_ref, xb_ref, scnt_ref, sbc_ref,
                    wla_ref, wlb_ref, wra_ref, wrb_ref, b768_ref,
                    apk_ref, bpk_ref, c128_ref, y1_ref, zrc_ref, rcp_ref):
    f32 = jnp.float32
    agg_a = pa_ref[0] + pa_ref[1]
    agg_b = pb_ref[0] + pb_ref[1]
    cnt = jnp.dot(agg_b, scnt_ref[...], preferred_element_type=f32)
    rcp = 1.0 / jnp.maximum(cnt, 1.0)
    rcpf = jnp.dot(rcp, sbc_ref[...], preferred_element_type=f32)
    h = (jnp.dot(agg_a * rcpf, wla_ref[...], preferred_element_type=f32)
         + jnp.dot(agg_b * rcpf, wlb_ref[...], preferred_element_type=f32)
         + jnp.dot(xa_ref[...], wra_ref[...], preferred_element_type=f32)
         + jnp.dot(xb_ref[...], wrb_ref[...], preferred_element_type=f32)
         + b768_ref[...])
    h = jnp.maximum(h, 0.0)
    y1_ref[...] = jnp.dot(h, apk_ref[...], preferred_element_type=f32)
    zrc_ref[...] = jnp.dot(h, bpk_ref[...],
                           preferred_element_type=f32) + c128_ref[...]
    rcp_ref[...] = rcpf


def _stage_mid(parts_a, parts_b, xap, xbp, scnt, sbc, wla, wlb, wra, wrb,
               b768, apk, bpk, c128):
    blk = pl.BlockSpec((_BP, 128), lambda i: (i, 0))
    full = lambda r, c: pl.BlockSpec((r, c), lambda i: (0, 0))
    return pl.pallas_call(
        _stage_mid_body,
        grid=(NPAD // _B,),
        in_specs=[
            pl.BlockSpec((2, _BP, 128), lambda i: (0, i, 0)),
            pl.BlockSpec((2, _BP, 128), lambda i: (0, i, 0)),
            blk, blk,
            full(128, 8), full(8, 128),
            full(128, 768), full(128, 768), full(128, 768), full(128, 768),
            full(1, 768), full(768, 128), full(768, 128), full(1, 128),
        ],
        out_specs=[blk, blk, blk],
        out_shape=[
            jax.ShapeDtypeStruct((_NP8, 128), jnp.float32),
            jax.ShapeDtypeStruct((_NP8, 128), jnp.float32),
            jax.ShapeDtypeStruct((_NP8, 128), jnp.float32),
        ],
    )(parts_a, parts_b, xap, xbp, scnt, sbc, wla, wlb, wra, wrb,
      b768, apk, bpk, c128)


def _stage_out_body(parts_ref, zrc_ref, rcp_ref, f20_ref, f21_ref,
                    b0_ref, b1_ref, o0_ref, o1_ref):
    f32 = jnp.float32
    agg = parts_ref[0] + parts_ref[1]
    z = jnp.maximum(agg * rcp_ref[...] + zrc_ref[...], 0.0)
    o0_ref[...] = jnp.dot(z, f20_ref[...],
                          preferred_element_type=f32) + b0_ref[...]
    o1_ref[...] = jnp.dot(z, f21_ref[...],
                          preferred_element_type=f32) + b1_ref[...]


def _stage_out(parts2, zrc, rcp, f20p, f21p, b0, b1):
    blk = pl.BlockSpec((_BP, 128), lambda i: (i, 0))
    return pl.pallas_call(
        _stage_out_body,
        grid=(NPAD // _B,),
        in_specs=[
            pl.BlockSpec((2, _BP, 128), lambda i: (0, i, 0)),
            blk, blk,
            pl.BlockSpec((128, 128), lambda i: (0, 0)),
            pl.BlockSpec((128, 128), lambda i: (0, 0)),
            pl.BlockSpec((1, 128), lambda i: (0, 0)),
            pl.BlockSpec((1, 128), lambda i: (0, 0)),
        ],
        out_specs=[blk, blk],
        out_shape=[
            jax.ShapeDtypeStruct((_NP8, 128), jnp.float32),
            jax.ShapeDtypeStruct((_NP8, 128), jnp.float32),
        ],
    )(parts2, zrc, rcp, f20p, f21p, b0, b1)


def kernel(x, edge_index, Wl1, bl1, Wr1, Wl2, bl2, Wr2,
           fc1_W, fc1_b, fc2_W, fc2_b):
    f32 = jnp.float32
    x24 = x.reshape(N, 24)
    padp = ((0, _NP8 - N // 8), (0, 0))
    xap = jnp.pad(x24[:, :16].reshape(N // 8, 128), padp)
    xbp = jnp.pad(jnp.concatenate(
        [x24[:, 16:], jnp.ones((N, 1), f32), jnp.zeros((N, 7), f32)],
        axis=1).reshape(N // 8, 128), padp)
    xp_a = xap.reshape(NPAD, 16)
    xp_b = xbp.reshape(NPAD, 16)
    src3 = edge_index[0].reshape(NW, NCH, CH)
    dst3 = edge_index[1].reshape(NW, NCH, CH)

    eye8 = jnp.eye(8, dtype=f32)
    eye12 = jnp.eye(12, dtype=f32)
    wl = jnp.einsum('st,gf->sgtf', eye12, Wl1).reshape(24, 96)
    wr = jnp.einsum('st,gf->sgtf', eye12, Wr1).reshape(24, 96)
    pad16 = lambda m: jnp.pad(m, ((0, 8), (0, 0)))
    wla = jnp.kron(eye8, wl[:16])
    wlb = jnp.kron(eye8, pad16(wl[16:]))
    wra = jnp.kron(eye8, wr[:16])
    wrb = jnp.kron(eye8, pad16(wr[16:]))
    b768 = jnp.tile(jnp.tile(bl1, 12), 8).reshape(1, 768)
    w1r = fc1_W.reshape(2, 12, 8, 8)
    a_f = jnp.einsum('gf,isfo->sgio', Wl2, w1r).reshape(96, 16)
    b_f = jnp.einsum('gf,isfo->sgio', Wr2, w1r).reshape(96, 16)
    apk = jnp.kron(eye8, a_f)
    bpk = jnp.kron(eye8, b_f)
    c16 = (jnp.einsum('f,isfo->io', bl2, w1r) + fc1_b).reshape(16)
    c128 = jnp.tile(c16, 8).reshape(1, 128)
    pad1216 = ((0, 8), (0, 4))
    f20p = jnp.kron(eye8, jnp.pad(fc2_W[0], pad1216))
    f21p = jnp.kron(eye8, jnp.pad(
        jnp.pad(fc2_W[1], ((8, 0), (0, 0))), ((0, 0), (0, 4))))
    b0 = jnp.tile(jnp.pad(fc2_b[0], (0, 4)), 8).reshape(1, 128)
    b1 = jnp.tile(jnp.pad(fc2_b[1], (0, 4)), 8).reshape(1, 128)
    scnt = jnp.zeros((128, 8), f32).at[jnp.arange(8) * 16 + 8,
                                       jnp.arange(8)].set(1.0)
    sbc = jnp.kron(eye8, jnp.ones((1, 16), f32))

    zr = jnp.zeros((RPT, 16), f32)
    parts1a = _seg_sum_16(xp_a, src3, dst3, zr)
    parts1b = _seg_sum_16(xp_b, src3, dst3, zr)
    y1p, zrc, rcp = _stage_mid(parts1a.reshape(2, _NP8, 128),
                               parts1b.reshape(2, _NP8, 128),
                               xap, xbp, scnt, sbc, wla, wlb, wra, wrb,
                               b768, apk, bpk, c128)
    y1 = y1p.reshape(NPAD, 16)
    parts2 = _seg_sum_16(y1, src3, dst3, zr)
    o0p, o1p = _stage_out(parts2.reshape(2, _NP8, 128), zrc, rcp,
                          f20p, f21p, b0, b1)
    o0 = o0p.reshape(NPAD, 16)[:N, :12]
    o1 = o1p.reshape(NPAD, 16)[:N, :12]
    return jnp.stack([o0, o1], axis=0)


# speedup vs baseline: 1.2520x; 1.0479x over previous
import functools

import jax
import jax.numpy as jnp
from jax import lax
from jax.experimental import pallas as pl
from jax.experimental.pallas import tpu as pltpu
from jax.experimental.pallas import tpu_sc as plsc

N = 50000
NPAD = 50048
E = 800000
NW = 32
EPW = E // NW
CH = 125
NCH = EPW // CH
RPT = NPAD // 16
RCH = 136
NRCH = RPT // RCH


NBUF = 10
LOOK = 5


def _seg_sum_body(table_hbm, src_hbm, dst_hbm, zero_hbm, out_hbm,
                  src_v, dst_v, bufs, acc, gsems, ssems, zsem, W):
    c = lax.axis_index("c")
    s = lax.axis_index("s")
    w = c * 16 + s

    pltpu.async_copy(zero_hbm, acc.at[pl.ds(s * RPT, RPT)], zsem)
    pltpu.sync_copy(src_hbm.at[w], src_v)
    pltpu.sync_copy(dst_hbm.at[w], dst_v)
    pltpu.make_async_copy(zero_hbm, acc.at[pl.ds(s * RPT, RPT)], zsem).wait()
    plsc.subcore_barrier()

    def _g_start(j, b):
        pltpu.async_copy(table_hbm.at[src_v.at[j]], bufs[b], gsems[b])

    def _g_wait(b):
        pltpu.make_async_copy(table_hbm.at[src_v.at[0]], bufs[b],
                              gsems[b]).wait()

    def _s_start(j, b):
        pltpu.async_copy(bufs[b], acc.at[dst_v.at[j]], ssems[b], add=True)

    def _s_wait(b):
        pltpu.make_async_copy(bufs[b], acc.at[dst_v.at[0]], ssems[b]).wait()

    for j in range(LOOK):
        _g_start(j, j)

    def _body(i, _):
        for off in range(NBUF):
            j = NBUF * i + off
            b = off
            bn = (off + LOOK) % NBUF
            _g_wait(b)
            _s_start(j, b)

            @pl.when(j + LOOK < NCH)
            def _():
                @pl.when(j >= LOOK)
                def _():
                    _s_wait(bn)
                _g_start(j + LOOK, bn)
        return 0
    lax.fori_loop(0, NCH // NBUF, _body, 0)
    for b in range(NBUF):
        _s_wait(b)
    plsc.subcore_barrier()

    r = s * RPT
    pltpu.async_copy(acc.at[pl.ds(r, RPT)], out_hbm.at[c, pl.ds(r, RPT)],
                     zsem)
    pltpu.make_async_copy(acc.at[pl.ds(r, RPT)],
                          out_hbm.at[c, pl.ds(r, RPT)], zsem).wait()


def _make_seg_sum(W):
    mesh = plsc.VectorSubcoreMesh(core_axis_name="c", subcore_axis_name="s")

    @functools.partial(
        pl.kernel,
        out_type=jax.ShapeDtypeStruct((2, NPAD, W), jnp.float32),
        mesh=mesh,
        compiler_params=pltpu.CompilerParams(use_tc_tiling_on_sc=False),
        scratch_types=[
            pltpu.VMEM((NCH, CH), jnp.int32),
            pltpu.VMEM((NCH, CH), jnp.int32),
        ] + [pltpu.VMEM((CH, W), jnp.float32) for _ in range(NBUF)]
        + [pltpu.VMEM_SHARED((NPAD, W), jnp.float32)]
        + [pltpu.SemaphoreType.DMA for _ in range(2 * NBUF + 1)],
    )
    def seg_sum(table_hbm, src_hbm, dst_hbm, zero_hbm, out_hbm,
                src_v, dst_v, *rest):
        bufs = rest[:NBUF]
        acc = rest[NBUF]
        gsems = rest[NBUF + 1:2 * NBUF + 1]
        ssems = rest[2 * NBUF + 1:3 * NBUF + 1]
        zsem = rest[3 * NBUF + 1]
        _seg_sum_body(table_hbm, src_hbm, dst_hbm, zero_hbm, out_hbm,
                      src_v, dst_v, bufs, acc, gsems, ssems, zsem, W)

    return seg_sum


_seg_sum_16 = _make_seg_sum(16)

_B = 2944
_BP = _B // 8
_NP8 = NPAD // 8


def _stage_mid_body(pa_ref, pb_ref, xa_ref, xb_ref, scnt_ref, sbc_ref,
                    wla_ref, wlb_ref, wra_ref, wrb_ref, b768_ref,
                    apk_ref, bpk_ref, c128_ref, y1_ref, zrc_ref, rcp_ref):
    f32 = jnp.float32
    agg_a = pa_ref[0] + pa_ref[1]
    agg_b = pb_ref[0] + pb_ref[1]
    cnt = jnp.dot(agg_b, scnt_ref[...], preferred_element_type=f32)
    rcp = 1.0 / jnp.maximum(cnt, 1.0)
    rcpf = jnp.dot(rcp, sbc_ref[...], preferred_element_type=f32)
    h = (jnp.dot(agg_a * rcpf, wla_ref[...], preferred_element_type=f32)
         + jnp.dot(agg_b * rcpf, wlb_ref[...], preferred_element_type=f32)
         + jnp.dot(xa_ref[...], wra_ref[...], preferred_element_type=f32)
         + jnp.dot(xb_ref[...], wrb_ref[...], preferred_element_type=f32)
         + b768_ref[...])
    h = jnp.maximum(h, 0.0)
    y1_ref[...] = jnp.dot(h, apk_ref[...], preferred_element_type=f32)
    zrc_ref[...] = jnp.dot(h, bpk_ref[...],
                           preferred_element_type=f32) + c128_ref[...]
    rcp_ref[...] = rcpf


def _stage_mid(parts_a, parts_b, xap, xbp, scnt, sbc, wla, wlb, wra, wrb,
               b768, apk, bpk, c128):
    blk = pl.BlockSpec((_BP, 128), lambda i: (i, 0))
    full = lambda r, c: pl.BlockSpec((r, c), lambda i: (0, 0))
    return pl.pallas_call(
        _stage_mid_body,
        grid=(NPAD // _B,),
        in_specs=[
            pl.BlockSpec((2, _BP, 128), lambda i: (0, i, 0)),
            pl.BlockSpec((2, _BP, 128), lambda i: (0, i, 0)),
            blk, blk,
            full(128, 8), full(8, 128),
            full(128, 768), full(128, 768), full(128, 768), full(128, 768),
            full(1, 768), full(768, 128), full(768, 128), full(1, 128),
        ],
        out_specs=[blk, blk, blk],
        out_shape=[
            jax.ShapeDtypeStruct((_NP8, 128), jnp.float32),
            jax.ShapeDtypeStruct((_NP8, 128), jnp.float32),
            jax.ShapeDtypeStruct((_NP8, 128), jnp.float32),
        ],
    )(parts_a, parts_b, xap, xbp, scnt, sbc, wla, wlb, wra, wrb,
      b768, apk, bpk, c128)


def _stage_out_body(parts_ref, zrc_ref, rcp_ref, f20_ref, f21_ref,
                    b0_ref, b1_ref, o0_ref, o1_ref):
    f32 = jnp.float32
    agg = parts_ref[0] + parts_ref[1]
    z = jnp.maximum(agg * rcp_ref[...] + zrc_ref[...], 0.0)
    o0_ref[...] = jnp.dot(z, f20_ref[...],
                          preferred_element_type=f32) + b0_ref[...]
    o1_ref[...] = jnp.dot(z, f21_ref[...],
                          preferred_element_type=f32) + b1_ref[...]


def _stage_out(parts2, zrc, rcp, f20p, f21p, b0, b1):
    blk = pl.BlockSpec((_BP, 128), lambda i: (i, 0))
    return pl.pallas_call(
        _stage_out_body,
        grid=(NPAD // _B,),
        in_specs=[
            pl.BlockSpec((2, _BP, 128), lambda i: (0, i, 0)),
            blk, blk,
            pl.BlockSpec((128, 128), lambda i: (0, 0)),
            pl.BlockSpec((128, 128), lambda i: (0, 0)),
            pl.BlockSpec((1, 128), lambda i: (0, 0)),
            pl.BlockSpec((1, 128), lambda i: (0, 0)),
        ],
        out_specs=[blk, blk],
        out_shape=[
            jax.ShapeDtypeStruct((_NP8, 128), jnp.float32),
            jax.ShapeDtypeStruct((_NP8, 128), jnp.float32),
        ],
    )(parts2, zrc, rcp, f20p, f21p, b0, b1)


def kernel(x, edge_index, Wl1, bl1, Wr1, Wl2, bl2, Wr2,
           fc1_W, fc1_b, fc2_W, fc2_b):
    f32 = jnp.float32
    x24 = x.reshape(N, 24)
    padp = ((0, _NP8 - N // 8), (0, 0))
    xap = jnp.pad(x24[:, :16].reshape(N // 8, 128), padp)
    xbp = jnp.pad(jnp.concatenate(
        [x24[:, 16:], jnp.ones((N, 1), f32), jnp.zeros((N, 7), f32)],
        axis=1).reshape(N // 8, 128), padp)
    xp_a = xap.reshape(NPAD, 16)
    xp_b = xbp.reshape(NPAD, 16)
    src3 = edge_index[0].reshape(NW, NCH, CH)
    dst3 = edge_index[1].reshape(NW, NCH, CH)

    eye8 = jnp.eye(8, dtype=f32)
    eye12 = jnp.eye(12, dtype=f32)
    wl = jnp.einsum('st,gf->sgtf', eye12, Wl1).reshape(24, 96)
    wr = jnp.einsum('st,gf->sgtf', eye12, Wr1).reshape(24, 96)
    pad16 = lambda m: jnp.pad(m, ((0, 8), (0, 0)))
    wla = jnp.kron(eye8, wl[:16])
    wlb = jnp.kron(eye8, pad16(wl[16:]))
    wra = jnp.kron(eye8, wr[:16])
    wrb = jnp.kron(eye8, pad16(wr[16:]))
    b768 = jnp.tile(jnp.tile(bl1, 12), 8).reshape(1, 768)
    w1r = fc1_W.reshape(2, 12, 8, 8)
    a_f = jnp.einsum('gf,isfo->sgio', Wl2, w1r).reshape(96, 16)
    b_f = jnp.einsum('gf,isfo->sgio', Wr2, w1r).reshape(96, 16)
    apk = jnp.kron(eye8, a_f)
    bpk = jnp.kron(eye8, b_f)
    c16 = (jnp.einsum('f,isfo->io', bl2, w1r) + fc1_b).reshape(16)
    c128 = jnp.tile(c16, 8).reshape(1, 128)
    pad1216 = ((0, 8), (0, 4))
    f20p = jnp.kron(eye8, jnp.pad(fc2_W[0], pad1216))
    f21p = jnp.kron(eye8, jnp.pad(
        jnp.pad(fc2_W[1], ((8, 0), (0, 0))), ((0, 0), (0, 4))))
    b0 = jnp.tile(jnp.pad(fc2_b[0], (0, 4)), 8).reshape(1, 128)
    b1 = jnp.tile(jnp.pad(fc2_b[1], (0, 4)), 8).reshape(1, 128)
    scnt = jnp.zeros((128, 8), f32).at[jnp.arange(8) * 16 + 8,
                                       jnp.arange(8)].set(1.0)
    sbc = jnp.kron(eye8, jnp.ones((1, 16), f32))

    zr = jnp.zeros((RPT, 16), f32)
    parts1a = _seg_sum_16(xp_a, src3, dst3, zr)
    parts1b = _seg_sum_16(xp_b, src3, dst3, zr)
    y1p, zrc, rcp = _stage_mid(parts1a.reshape(2, _NP8, 128),
                               parts1b.reshape(2, _NP8, 128),
                               xap, xbp, scnt, sbc, wla, wlb, wra, wrb,
                               b768, apk, bpk, c128)
    y1 = y1p.reshape(NPAD, 16)
    parts2 = _seg_sum_16(y1, src3, dst3, zr)
    o0p, o1p = _stage_out(parts2.reshape(2, _NP8, 128), zrc, rcp,
                          f20p, f21p, b0, b1)
    o0 = o0p.reshape(NPAD, 16)[:N, :12]
    o1 = o1p.reshape(NPAD, 16)[:N, :12]
    return jnp.stack([o0, o1], axis=0)
